# baseline (speedup 1.0000x reference)
import functools

import jax
import jax.numpy as jnp
from jax import lax
from jax.experimental import pallas as pl
from jax.experimental.pallas import tpu as pltpu
from jax.experimental.pallas import tpu_sc as plsc

N = 10000
D = 128
E = 320000

NC = 2
NS = 16
NW = NC * NS
B = 64
NB = 4
OFF = NB // 2
L = 16
E_PAD = 327680
ROWS = E_PAD // (NS * B)
N_PAD = 10112
RPT = N_PAD // NS

_mesh = plsc.VectorSubcoreMesh(core_axis_name="c", subcore_axis_name="s")


def _make_segsum(with_counts):
    out_type = [jax.ShapeDtypeStruct((N_PAD, D), jnp.float32)]
    scratch = (
        [pltpu.VMEM((2, B), jnp.int32) for _ in range(NB)]
        + [pltpu.VMEM((B, D), jnp.float32) for _ in range(NB)]
        + [pltpu.VMEM_SHARED((N_PAD, D), jnp.float32)]
        + [pltpu.SemaphoreType.DMA for _ in range(NB)]
        + [pltpu.SemaphoreType.DMA for _ in range(NB)]
    )
    if with_counts:
        out_type.append(jax.ShapeDtypeStruct((NS, N_PAD), jnp.float32))
        scratch.append(pltpu.VMEM((N_PAD,), jnp.float32))

    @functools.partial(pl.kernel, out_type=out_type, mesh=_mesh,
                       scratch_types=scratch,
                       compiler_params=pltpu.CompilerParams(
                           needs_layout_passes=False))
    def segsum(table, edge0, zeros, out, *rest):
        if with_counts:
            cnt_out = rest[0]
            rest = rest[1:]
        ix = rest[0:NB]
        rv = rest[NB:2 * NB]
        acc = rest[2 * NB]
        gs = rest[2 * NB + 1:3 * NB + 1]
        ss = rest[3 * NB + 1:4 * NB + 1]
        cnt_v = rest[4 * NB + 1] if with_counts else None
        c = lax.axis_index("c")
        s = lax.axis_index("s")

        @pl.when(c == 0)
        def _():
            pltpu.sync_copy(zeros.at[pl.ds(s * RPT, RPT)],
                            acc.at[pl.ds(s * RPT, RPT)])
            if with_counts:
                def zc(i, carry):
                    cnt_v[pl.ds(i * L, L)] = jnp.zeros((L,), jnp.float32)
                    return carry
                lax.fori_loop(0, N_PAD // L, zc, 0)
        plsc.subcore_barrier()

        ones_l = jnp.full((L,), 1.0, jnp.float32)

        def count(ixj):
            if with_counts:
                for k in range(B // L):
                    plsc.addupdate_scatter(
                        cnt_v, [ixj[1, pl.ds(k * L, L)]], ones_l)

        def run(edge_r, rows):
            def start_gather(j, ch):
                pltpu.sync_copy(edge_r.at[s, ch], ix[j])
                pltpu.async_copy(table.at[ix[j].at[0]], rv[j], gs[j])

            def wait_gather(j):
                pltpu.make_async_copy(table.at[ix[j].at[0]], rv[j], gs[j]).wait()

            def start_scatter(j):
                pltpu.async_copy(rv[j], acc.at[ix[j].at[1]], ss[j], add=True)

            def wait_scatter(j):
                pltpu.make_async_copy(rv[j], acc.at[ix[j].at[1]], ss[j]).wait()

            for j in range(OFF):
                start_gather(j, j)

            def slot(i, j):
                ch = NB * i + j
                wait_gather(j)
                start_scatter(j)
                count(ix[j])
                jq = (j + OFF) % NB
                @pl.when(ch >= OFF)
                def _():
                    wait_scatter(jq)
                @pl.when(ch + OFF < rows)
                def _():
                    start_gather(jq, ch + OFF)

            def ring(i, carry):
                for j in range(NB):
                    slot(i, j)
                return carry

            lax.fori_loop(0, rows // NB, ring, 0)

            for k in range(OFF):
                wait_scatter((rows - OFF + k) % NB)

        @pl.when(c == 0)
        def _():
            run(edge0, ROWS)

        plsc.subcore_barrier()

        @pl.when(c == 0)
        def _():
            pltpu.sync_copy(acc.at[pl.ds(s * RPT, RPT)],
                            out.at[pl.ds(s * RPT, RPT)])
            if with_counts:
                pltpu.sync_copy(cnt_v, cnt_out.at[s])

    return segsum


_sc_segsum_cnt = _make_segsum(True)
_sc_segsum = _make_segsum(False)


def _matmul_t_body(x_ref, w_ref, o_ref):
    o_ref[:] = lax.dot_general(x_ref[:], w_ref[:], (((1,), (1,)), ((), ())),
                               preferred_element_type=jnp.float32)


def _tc_matmul_t(x, w):
    return pl.pallas_call(
        _matmul_t_body,
        out_shape=jax.ShapeDtypeStruct((x.shape[0], w.shape[0]), jnp.float32),
    )(x, w)


def _combine_body(relu, seg_ref, cnt_ref, x_ref, wr_ref, bl_ref, o_ref):
    cnt = jnp.reshape(jnp.sum(cnt_ref[:], axis=0), (N, 1))
    mean = seg_ref[:] / jnp.maximum(cnt, 1.0)
    root = lax.dot_general(x_ref[:], wr_ref[:], (((1,), (1,)), ((), ())),
                           preferred_element_type=jnp.float32)
    o = mean + bl_ref[:] + root
    if relu:
        o = jnp.maximum(o, 0.0)
    o_ref[:] = o


def _tc_combine(seg, cnt, x, wr, bl, relu):
    return pl.pallas_call(
        functools.partial(_combine_body, relu),
        out_shape=jax.ShapeDtypeStruct((N, D), jnp.float32),
    )(seg, cnt, x, wr, bl)


def kernel(x, edge_index, W1l, b1l, W1r, W2l, b2l, W2r):
    src = edge_index[0]
    dst = edge_index[1]
    pad = E_PAD - E
    src_p = jnp.concatenate([src, jnp.zeros((pad,), jnp.int32)])
    dst_p = jnp.concatenate([dst, jnp.full((pad,), N, jnp.int32)])
    edge0 = jnp.stack([src_p.reshape(NS, ROWS, B),
                       dst_p.reshape(NS, ROWS, B)], axis=2)

    zeros = jnp.zeros((N_PAD, D), jnp.float32)

    t1 = _tc_matmul_t(x, W1l)
    seg1, cnt_raw = _sc_segsum_cnt(t1, edge0, zeros)
    cnt = cnt_raw[:, :N]
    h = _tc_combine(seg1[:N], cnt, x, W1r, b1l.reshape(1, D), relu=True)

    t2 = _tc_matmul_t(h, W2l)
    (seg2,) = _sc_segsum(t2, edge0, zeros)
    out = _tc_combine(seg2[:N], cnt, h, W2r, b2l.reshape(1, D), relu=False)
    return out


# speedup vs baseline: 1.2862x; 1.2862x over previous
import functools

import jax
import jax.numpy as jnp
from jax import lax
from jax.experimental import pallas as pl
from jax.experimental.pallas import tpu as pltpu
from jax.experimental.pallas import tpu_sc as plsc

N = 10000
D = 128
E = 320000

NC = 2
NS = 16
NW = NC * NS
B = 64
NB = 4
OFF = NB // 2
L = 16
E_PAD = 327680
ROWS = E_PAD // (NW * B)
N_PAD = 10112
RPT = N_PAD // NS

_mesh = plsc.VectorSubcoreMesh(core_axis_name="c", subcore_axis_name="s")


def _make_segsum(with_counts):
    out_type = [jax.ShapeDtypeStruct((NC, N_PAD, D), jnp.float32)]
    scratch = (
        [pltpu.VMEM((2, B), jnp.int32) for _ in range(NB)]
        + [pltpu.VMEM((B, D), jnp.float32) for _ in range(NB)]
        + [pltpu.VMEM_SHARED((N_PAD, D), jnp.float32)]
        + [pltpu.SemaphoreType.DMA for _ in range(NB)]
        + [pltpu.SemaphoreType.DMA for _ in range(NB)]
    )
    if with_counts:
        out_type.append(jax.ShapeDtypeStruct((NW, N_PAD), jnp.float32))
        scratch.append(pltpu.VMEM((N_PAD,), jnp.float32))

    @functools.partial(pl.kernel, out_type=out_type, mesh=_mesh,
                       scratch_types=scratch,
                       compiler_params=pltpu.CompilerParams(
                           needs_layout_passes=False))
    def segsum(table, edge_r, zeros, out, *rest):
        if with_counts:
            cnt_out = rest[0]
            rest = rest[1:]
        ix = rest[0:NB]
        rv = rest[NB:2 * NB]
        acc = rest[2 * NB]
        gs = rest[2 * NB + 1:3 * NB + 1]
        ss = rest[3 * NB + 1:4 * NB + 1]
        cnt_v = rest[4 * NB + 1] if with_counts else None
        c = lax.axis_index("c")
        s = lax.axis_index("s")

        pltpu.sync_copy(zeros.at[pl.ds(s * RPT, RPT)],
                        acc.at[pl.ds(s * RPT, RPT)])
        if with_counts:
            def zc(i, carry):
                cnt_v[pl.ds(i * L, L)] = jnp.zeros((L,), jnp.float32)
                return carry
            lax.fori_loop(0, N_PAD // L, zc, 0)
        plsc.subcore_barrier()

        ones_l = jnp.full((L,), 1.0, jnp.float32)

        def count(ixj):
            if with_counts:
                for k in range(B // L):
                    plsc.addupdate_scatter(
                        cnt_v, [ixj[1, pl.ds(k * L, L)]], ones_l)

        def run(edge_r, rows):
            w = c * NS + s

            def start_gather(j, ch):
                pltpu.sync_copy(edge_r.at[w, ch], ix[j])
                pltpu.async_copy(table.at[ix[j].at[0]], rv[j], gs[j])

            def wait_gather(j):
                pltpu.make_async_copy(table.at[ix[j].at[0]], rv[j], gs[j]).wait()

            def start_scatter(j):
                pltpu.async_copy(rv[j], acc.at[ix[j].at[1]], ss[j], add=True)

            def wait_scatter(j):
                pltpu.make_async_copy(rv[j], acc.at[ix[j].at[1]], ss[j]).wait()

            for j in range(OFF):
                start_gather(j, j)

            def slot(i, j):
                ch = NB * i + j
                wait_gather(j)
                start_scatter(j)
                count(ix[j])
                jq = (j + OFF) % NB
                @pl.when(ch >= OFF)
                def _():
                    wait_scatter(jq)
                @pl.when(ch + OFF < rows)
                def _():
                    start_gather(jq, ch + OFF)

            def ring(i, carry):
                for j in range(NB):
                    slot(i, j)
                return carry

            lax.fori_loop(0, rows // NB, ring, 0)

            for k in range(OFF):
                wait_scatter((rows - OFF + k) % NB)

        run(edge_r, ROWS)

        plsc.subcore_barrier()
        pltpu.sync_copy(acc.at[pl.ds(s * RPT, RPT)],
                        out.at[c, pl.ds(s * RPT, RPT)])
        if with_counts:
            pltpu.sync_copy(cnt_v, cnt_out.at[c * NS + s])

    return segsum


_sc_segsum_cnt = _make_segsum(True)
_sc_segsum = _make_segsum(False)


def _matmul_t_body(x_ref, w_ref, o_ref):
    o_ref[:] = lax.dot_general(x_ref[:], w_ref[:], (((1,), (1,)), ((), ())),
                               preferred_element_type=jnp.float32)


def _tc_matmul_t(x, w):
    return pl.pallas_call(
        _matmul_t_body,
        out_shape=jax.ShapeDtypeStruct((x.shape[0], w.shape[0]), jnp.float32),
    )(x, w)


def _combine_body(relu, seg_ref, cnt_ref, x_ref, wr_ref, bl_ref, o_ref):
    cnt = jnp.reshape(jnp.sum(cnt_ref[:], axis=0), (N, 1))
    mean = (seg_ref[0] + seg_ref[1]) / jnp.maximum(cnt, 1.0)
    root = lax.dot_general(x_ref[:], wr_ref[:], (((1,), (1,)), ((), ())),
                           preferred_element_type=jnp.float32)
    o = mean + bl_ref[:] + root
    if relu:
        o = jnp.maximum(o, 0.0)
    o_ref[:] = o


def _tc_combine(seg, cnt, x, wr, bl, relu):
    return pl.pallas_call(
        functools.partial(_combine_body, relu),
        out_shape=jax.ShapeDtypeStruct((N, D), jnp.float32),
    )(seg, cnt, x, wr, bl)


def kernel(x, edge_index, W1l, b1l, W1r, W2l, b2l, W2r):
    src = edge_index[0]
    dst = edge_index[1]
    pad = E_PAD - E
    src_p = jnp.concatenate([src, jnp.zeros((pad,), jnp.int32)])
    dst_p = jnp.concatenate([dst, jnp.full((pad,), N, jnp.int32)])
    edge_r = jnp.stack([src_p.reshape(NW, ROWS, B),
                        dst_p.reshape(NW, ROWS, B)], axis=2)

    zeros = jnp.zeros((N_PAD, D), jnp.float32)

    t1 = _tc_matmul_t(x, W1l)
    seg1, cnt_raw = _sc_segsum_cnt(t1, edge_r, zeros)
    cnt = cnt_raw[:, :N]
    h = _tc_combine(seg1[:, :N], cnt, x, W1r, b1l.reshape(1, D), relu=True)

    t2 = _tc_matmul_t(h, W2l)
    (seg2,) = _sc_segsum(t2, edge_r, zeros)
    out = _tc_combine(seg2[:, :N], cnt, h, W2r, b2l.reshape(1, D), relu=False)
    return out
